# Initial kernel scaffold; baseline (speedup 1.0000x reference)
import jax
import jax.numpy as jnp
from jax.experimental import pallas as pl


def kernel(x, edge_index, train_edge_id, W_ppc, b_ppc, W1, a1_src, a1_dst, b1, W2, a2_src, a2_dst, b2, lin1_W, lin1_b, lin2_W, lin2_b, fc2_W, fc2_b):
    raise NotImplementedError("write your pallas kernel here")


# speedup vs baseline: 2.8127x; 2.8127x over previous
import functools
import jax
import jax.numpy as jnp
import numpy as np
from jax import lax
from jax.experimental import pallas as pl
from jax.experimental.pallas import tpu as pltpu

N = 50000
E = 800000
L = 50
NPAD = 50048
EP = 860160
NT = 50008


def _lrelu(v):
    return jnp.where(v >= 0, v, 0.2 * v)


def _k1_body(xb, wmat, bppc, wbig, h0, h1o, h2, h3, h4, s1o, d1o):
    Bn = xb.shape[0]
    xp = jnp.pad(xb[...], ((0, 0), (1, 1), (0, 0)))
    xcat = jnp.concatenate(
        [xp[:, 0:L, :], xp[:, 1:L + 1, :], xp[:, 2:L + 2, :]], axis=2)
    a = xcat.reshape(Bn * L, 48)
    z = jnp.dot(a, wmat[...], preferred_element_type=jnp.float32)
    z = jnp.maximum(z.reshape(Bn, L, 256) + bppc[...], 0.0)
    p = jnp.max(z, axis=1)
    m = jnp.dot(p, wbig[...], preferred_element_type=jnp.float32)
    for i, o in enumerate((h0, h1o, h2, h3, h4)):
        o[...] = m[:, 16 * i:16 * i + 16]
    s1o[...] = m[:, 80:96]
    d1o[...] = m[:, 96:112]


def _conv_proj1(xpad, wmat, bppc, wbig):
    Bn = 128
    grid = NPAD // Bn
    outs = [jax.ShapeDtypeStruct((NPAD, 16), jnp.float32)] * 7
    ospec = pl.BlockSpec((Bn, 16), lambda i: (i, 0))
    return pl.pallas_call(
        _k1_body,
        grid=(grid,),
        in_specs=[
            pl.BlockSpec((Bn, L, 16), lambda i: (i, 0, 0)),
            pl.BlockSpec((48, 256), lambda i: (0, 0)),
            pl.BlockSpec((1, 256), lambda i: (0, 0)),
            pl.BlockSpec((256, 112), lambda i: (0, 0)),
        ],
        out_specs=[ospec] * 7,
        out_shape=outs,
    )(xpad, wmat, bppc, wbig)


def _k2_body(a0, a1, a2, a3, a4, den, b1, e8, wsd2,
             x0, x1o, x2, x3, x4, s2o, d2o):
    parts = [r[...][0] + r[...][1] for r in (a0, a1, a2, a3, a4)]
    agg = jnp.concatenate(parts, axis=1)
    den16 = den[...][0] + den[...][1]
    denexp = jnp.dot(den16, e8[...], preferred_element_type=jnp.float32)
    xv = jnp.maximum(agg / (denexp + 1e-16) + b1[...], 0.0)
    m2 = jnp.dot(xv, wsd2[...], preferred_element_type=jnp.float32)
    for i, o in enumerate((x0, x1o, x2, x3, x4)):
        o[...] = xv[:, 16 * i:16 * i + 16]
    s2o[...] = m2[:, 0:16]
    d2o[...] = m2[:, 16:32]


def _proj2(aggs, den, b1, e8, wsd2):
    Bn = 2000
    grid = N // Bn
    aspec = pl.BlockSpec((2, Bn, 16), lambda i: (0, i, 0))
    ospec = pl.BlockSpec((Bn, 16), lambda i: (i, 0))
    return pl.pallas_call(
        _k2_body,
        grid=(grid,),
        in_specs=[aspec] * 5 + [
            aspec,
            pl.BlockSpec((1, 80), lambda i: (0, 0)),
            pl.BlockSpec((16, 80), lambda i: (0, 0)),
            pl.BlockSpec((80, 32), lambda i: (0, 0)),
        ],
        out_specs=[ospec] * 7,
        out_shape=[jax.ShapeDtypeStruct((N, 16), jnp.float32)] * 7,
    )(*aggs, den, b1, e8, wsd2)


def _k3_body(a0, a1, a2, a3, a4, den, w2, b2, l1w, l1b, l2w, l2b, yo):
    parts = [r[...][0] + r[...][1] for r in (a0, a1, a2, a3, a4)]
    agg = jnp.concatenate(parts, axis=1)
    den16 = den[...][0] + den[...][1]
    dcol = den16[:, 0:1] + 1e-16
    zn = agg / dcol
    z2 = jnp.dot(zn, w2[...], preferred_element_type=jnp.float32) + b2[...]
    t = jnp.maximum(
        jnp.dot(z2, l1w[...], preferred_element_type=jnp.float32) + l1b[...], 0.0)
    yo[...] = jnp.dot(t, l2w[...], preferred_element_type=jnp.float32) + l2b[...]


def _tail(aggs, den, w2, b2, l1w, l1b, l2w, l2b):
    Bn = 1000
    grid = N // Bn
    aspec = pl.BlockSpec((2, Bn, 16), lambda i: (0, i, 0))
    return pl.pallas_call(
        _k3_body,
        grid=(grid,),
        in_specs=[aspec] * 6 + [
            pl.BlockSpec((80, 512), lambda i: (0, 0)),
            pl.BlockSpec((1, 512), lambda i: (0, 0)),
            pl.BlockSpec((512, 512), lambda i: (0, 0)),
            pl.BlockSpec((1, 512), lambda i: (0, 0)),
            pl.BlockSpec((512, 512), lambda i: (0, 0)),
            pl.BlockSpec((1, 512), lambda i: (0, 0)),
        ],
        out_specs=pl.BlockSpec((Bn, 512), lambda i: (i, 0)),
        out_shape=jax.ShapeDtypeStruct((N, 512), jnp.float32),
    )(*aggs, den, w2, b2, l1w, l1b, l2w, l2b)


def _k4_body(prod, fw, fb, oo):
    oo[...] = jnp.dot(prod[...], fw[...],
                      preferred_element_type=jnp.float32) + fb[...]


def _fc2(prod, fw, fb):
    Bp = 2048
    grid = prod.shape[0] // Bp
    return pl.pallas_call(
        _k4_body,
        grid=(grid,),
        in_specs=[
            pl.BlockSpec((Bp, 512), lambda i: (i, 0)),
            pl.BlockSpec((512, 128), lambda i: (0, 0)),
            pl.BlockSpec((1, 128), lambda i: (0, 0)),
        ],
        out_specs=pl.BlockSpec((Bp, 128), lambda i: (i, 0)),
        out_shape=jax.ShapeDtypeStruct((prod.shape[0], 128), jnp.float32),
    )(prod, fw, fb)


def _edge_phase(src, dst, s_tab, d_tab, h_tabs):
    ex = jnp.exp(_lrelu(s_tab[src] + d_tab[dst]))
    den = jax.ops.segment_sum(ex, dst, num_segments=NT)[:N]
    aggs = []
    npat = np.arange(16)
    for p, ht in enumerate(h_tabs):
        pat = ((16 * p + npat) // 10).astype(np.int32)
        mult = ex[:, pat] if ht is not None else ex
        aggs.append(jax.ops.segment_sum(mult * ht[src], dst, num_segments=NT)[:N])
    z = jnp.zeros_like(den)
    den2 = jnp.stack([den, z])
    return den2, [jnp.stack([a, z]) for a in aggs]


def _edge_phase1(src, dst, s_tab, d_tab, h_tabs):
    ex = jnp.exp(_lrelu(s_tab[src] + d_tab[dst]))
    den = jax.ops.segment_sum(ex, dst, num_segments=NT)[:N]
    aggs = []
    npat = np.arange(16)
    for p, ht in enumerate(h_tabs):
        pat = ((16 * p + npat) // 10).astype(np.int32)
        mult = ex[:, pat]
        aggs.append(jax.ops.segment_sum(mult * ht[src], dst, num_segments=NT)[:N])
    z = jnp.zeros_like(den)
    return (jnp.stack([den, z]),
            [jnp.stack([a, z]) for a in aggs])


def _edge_phase2(src, dst, s_tab, d_tab, h_tabs):
    ex = jnp.exp(_lrelu(s_tab[src] + d_tab[dst]))
    den = jax.ops.segment_sum(ex, dst, num_segments=NT)[:N]
    aggs = [jax.ops.segment_sum(ex * ht[src], dst, num_segments=NT)[:N]
            for ht in h_tabs]
    z = jnp.zeros_like(den)
    return (jnp.stack([den, z]),
            [jnp.stack([a, z]) for a in aggs])


def _pair_gather(y, ei0, ei1, tid):
    s = ei0[tid]
    d = ei1[tid]
    return y[s] * y[d]


def kernel(x, edge_index, train_edge_id, W_ppc, b_ppc, W1, a1_src, a1_dst, b1,
           W2, a2_src, a2_dst, b2, lin1_W, lin1_b, lin2_W, lin2_b, fc2_W, fc2_b):
    f32 = jnp.float32
    wmat = jnp.transpose(W_ppc, (2, 1, 0)).reshape(48, 256)
    u1 = jnp.einsum('chj,hj->ch', W1.reshape(256, 8, 10), a1_src)
    v1 = jnp.einsum('chj,hj->ch', W1.reshape(256, 8, 10), a1_dst)
    wbig = jnp.concatenate([W1, u1, u1, v1, v1], axis=1)
    u2 = W2 @ a2_src[0]
    v2 = W2 @ a2_dst[0]
    wsd2 = jnp.concatenate([jnp.tile(u2[:, None], (1, 16)),
                            jnp.tile(v2[:, None], (1, 16))], axis=1)
    lane = np.arange(16)[:, None]
    feat = np.arange(80)[None, :]
    e8 = jnp.asarray((lane == feat // 10).astype(np.float32))
    fw = jnp.pad(fc2_W, ((0, 0), (0, 121)))
    fb = jnp.pad(fc2_b, (0, 121)).reshape(1, 128)

    loops = jnp.arange(N, dtype=jnp.int32)
    npad = EP - E - N
    src = jnp.concatenate([edge_index[0], loops,
                           jnp.zeros((npad,), jnp.int32)])
    dst = jnp.concatenate([edge_index[1], loops,
                           jnp.full((npad,), N, jnp.int32)])

    xpad = jnp.pad(x, ((0, NPAD - N), (0, 0), (0, 0)))

    t1 = _conv_proj1(xpad, wmat, b_ppc.reshape(1, 256), wbig)
    h1_tabs = [jnp.pad(t[:N], ((0, 8), (0, 0))) for t in t1[:5]]
    s1t = jnp.pad(t1[5][:N], ((0, 8), (0, 0)))
    d1t = jnp.pad(t1[6][:N], ((0, 8), (0, 0)))

    den1, agg1 = _edge_phase1(src, dst, s1t, d1t, h1_tabs)

    t2 = _proj2(agg1, den1, b1.reshape(1, 80), e8, wsd2)
    x1_tabs = [jnp.pad(t, ((0, 8), (0, 0))) for t in t2[:5]]
    s2t = jnp.pad(t2[5], ((0, 8), (0, 0)))
    d2t = jnp.pad(t2[6], ((0, 8), (0, 0)))

    den2, agg2 = _edge_phase2(src, dst, s2t, d2t, x1_tabs)

    y = _tail(agg2, den2, W2, b2.reshape(1, 512), lin1_W,
              lin1_b.reshape(1, 512), lin2_W, lin2_b.reshape(1, 512))

    prod = _pair_gather(y, edge_index[0], edge_index[1], train_edge_id)
    out = _fc2(prod, fw, fb)
    return out[:, :7]


# speedup vs baseline: 19.6195x; 6.9754x over previous
import functools
import jax
import jax.numpy as jnp
import numpy as np
from jax import lax
from jax.experimental import pallas as pl
from jax.experimental.pallas import tpu as pltpu
from jax.experimental.pallas import tpu_sc as plsc

N = 50000
E = 800000
L = 50
NPAD = 50048
NT = NPAD
EP = 860160
EW = EP // 32
EC = 1280
NCH = EW // EC
RSUB = NT // 16


def _lrelu(v):
    return jnp.where(v >= 0, v, 0.2 * v)


def _k1_body(xb, wmat, bppc, wbig, h0, h1o, h2, h3, h4, s1o, d1o):
    Bn = xb.shape[0]
    xp = jnp.pad(xb[...], ((0, 0), (1, 1), (0, 0)))
    xcat = jnp.concatenate(
        [xp[:, 0:L, :], xp[:, 1:L + 1, :], xp[:, 2:L + 2, :]], axis=2)
    a = xcat.reshape(Bn * L, 48)
    z = jnp.dot(a, wmat[...], preferred_element_type=jnp.float32)
    z = jnp.maximum(z.reshape(Bn, L, 256) + bppc[...], 0.0)
    p = jnp.max(z, axis=1)
    m = jnp.dot(p, wbig[...], preferred_element_type=jnp.float32)
    for i, o in enumerate((h0, h1o, h2, h3, h4)):
        o[...] = m[:, 16 * i:16 * i + 16]
    s1o[...] = m[:, 80:96]
    d1o[...] = m[:, 96:112]


def _conv_proj1(xpad, wmat, bppc, wbig):
    Bn = 128
    grid = NPAD // Bn
    outs = [jax.ShapeDtypeStruct((NPAD, 16), jnp.float32)] * 7
    ospec = pl.BlockSpec((Bn, 16), lambda i: (i, 0))
    return pl.pallas_call(
        _k1_body,
        grid=(grid,),
        in_specs=[
            pl.BlockSpec((Bn, L, 16), lambda i: (i, 0, 0)),
            pl.BlockSpec((48, 256), lambda i: (0, 0)),
            pl.BlockSpec((1, 256), lambda i: (0, 0)),
            pl.BlockSpec((256, 112), lambda i: (0, 0)),
        ],
        out_specs=[ospec] * 7,
        out_shape=outs,
    )(xpad, wmat, bppc, wbig)


def _k2_body(a0, a1, a2, a3, a4, den, b1, e8, wsd2,
             x0, x1o, x2, x3, x4, s2o, d2o):
    parts = [r[...][0] + r[...][1] for r in (a0, a1, a2, a3, a4)]
    agg = jnp.concatenate(parts, axis=1)
    den16 = den[...][0] + den[...][1]
    denexp = jnp.dot(den16, e8[...], preferred_element_type=jnp.float32)
    xv = jnp.maximum(agg / (denexp + 1e-16) + b1[...], 0.0)
    m2 = jnp.dot(xv, wsd2[...], preferred_element_type=jnp.float32)
    for i, o in enumerate((x0, x1o, x2, x3, x4)):
        o[...] = xv[:, 16 * i:16 * i + 16]
    s2o[...] = m2[:, 0:16]
    d2o[...] = m2[:, 16:32]


def _proj2(aggs, den, b1, e8, wsd2):
    Bn = 2000
    grid = N // Bn
    aspec = pl.BlockSpec((2, Bn, 16), lambda i: (0, i, 0))
    ospec = pl.BlockSpec((Bn, 16), lambda i: (i, 0))
    return pl.pallas_call(
        _k2_body,
        grid=(grid,),
        in_specs=[aspec] * 5 + [
            aspec,
            pl.BlockSpec((1, 80), lambda i: (0, 0)),
            pl.BlockSpec((16, 80), lambda i: (0, 0)),
            pl.BlockSpec((80, 32), lambda i: (0, 0)),
        ],
        out_specs=[ospec] * 7,
        out_shape=[jax.ShapeDtypeStruct((N, 16), jnp.float32)] * 7,
    )(*aggs, den, b1, e8, wsd2)


def _k3_body(a0, a1, a2, a3, a4, den, w2, b2, l1w, l1b, l2w, l2b, yo):
    parts = [r[...][0] + r[...][1] for r in (a0, a1, a2, a3, a4)]
    agg = jnp.concatenate(parts, axis=1)
    den16 = den[...][0] + den[...][1]
    dcol = den16[:, 0:1] + 1e-16
    zn = agg / dcol
    z2 = jnp.dot(zn, w2[...], preferred_element_type=jnp.float32) + b2[...]
    t = jnp.maximum(
        jnp.dot(z2, l1w[...], preferred_element_type=jnp.float32) + l1b[...], 0.0)
    yo[...] = jnp.dot(t, l2w[...], preferred_element_type=jnp.float32) + l2b[...]


def _tail(aggs, den, w2, b2, l1w, l1b, l2w, l2b):
    Bn = 1000
    grid = N // Bn
    aspec = pl.BlockSpec((2, Bn, 16), lambda i: (0, i, 0))
    return pl.pallas_call(
        _k3_body,
        grid=(grid,),
        in_specs=[aspec] * 6 + [
            pl.BlockSpec((80, 512), lambda i: (0, 0)),
            pl.BlockSpec((1, 512), lambda i: (0, 0)),
            pl.BlockSpec((512, 512), lambda i: (0, 0)),
            pl.BlockSpec((1, 512), lambda i: (0, 0)),
            pl.BlockSpec((512, 512), lambda i: (0, 0)),
            pl.BlockSpec((1, 512), lambda i: (0, 0)),
        ],
        out_specs=pl.BlockSpec((Bn, 512), lambda i: (i, 0)),
        out_shape=jax.ShapeDtypeStruct((N, 512), jnp.float32),
    )(*aggs, den, w2, b2, l1w, l1b, l2w, l2b)


def _k4_body(prod, fw, fb, oo):
    oo[...] = jnp.dot(prod[...], fw[...],
                      preferred_element_type=jnp.float32) + fb[...]


def _fc2(prod, fw, fb):
    Bp = 2048
    grid = prod.shape[0] // Bp
    return pl.pallas_call(
        _k4_body,
        grid=(grid,),
        in_specs=[
            pl.BlockSpec((Bp, 512), lambda i: (i, 0)),
            pl.BlockSpec((512, 128), lambda i: (0, 0)),
            pl.BlockSpec((1, 128), lambda i: (0, 0)),
        ],
        out_specs=pl.BlockSpec((Bp, 128), lambda i: (i, 0)),
        out_shape=jax.ShapeDtypeStruct((prod.shape[0], 128), jnp.float32),
    )(prod, fw, fb)


_MESH = dict(core_axis_name="c", subcore_axis_name="s")
_F32 = jnp.float32


def _worker_id():
    return lax.axis_index("s") * 2 + lax.axis_index("c")


def _zero_acc(zb, acc):
    sid = lax.axis_index("s")

    @pl.loop(0, EC)
    def _z(j):
        zb[j] = jnp.zeros((16,), _F32)

    r0 = sid * RSUB
    off = 0
    while off < RSUB:
        n = min(EC, RSUB - off)
        pltpu.sync_copy(zb.at[pl.ds(0, n)], acc.at[pl.ds(r0 + off, n)])
        off += n
    plsc.subcore_barrier()


def _flush_acc(acc, out_h):
    plsc.subcore_barrier()
    cid = lax.axis_index("c")
    sid = lax.axis_index("s")
    r0 = sid * RSUB
    pltpu.sync_copy(acc.at[pl.ds(r0, RSUB)], out_h.at[cid, pl.ds(r0, RSUB)])


def _sc_pass_a(src, dst, s_tab, d_tab):
    @functools.partial(
        pl.kernel,
        mesh=plsc.VectorSubcoreMesh(**_MESH),
        compiler_params=pltpu.CompilerParams(use_tc_tiling_on_sc=False),
        out_type=[jax.ShapeDtypeStruct((EP, 16), _F32),
                  jax.ShapeDtypeStruct((2, NT, 16), _F32)],
        scratch_types=[
            pltpu.VMEM((EC,), jnp.int32), pltpu.VMEM((EC,), jnp.int32),
            pltpu.VMEM((EC, 16), _F32), pltpu.VMEM((EC, 16), _F32),
            pltpu.VMEM((EC, 16), _F32),
            pltpu.VMEM_SHARED((NT, 16), _F32),
            pltpu.SemaphoreType.DMA, pltpu.SemaphoreType.DMA,
        ],
    )
    def k(src_h, dst_h, s_h, d_h, ex_h, den_h,
          sidx, didx, gs, gd, exb, acc, sem1, sem2):
        wid = _worker_id()
        _zero_acc(exb, acc)

        @pl.loop(0, NCH)
        def _g(g):
            base = wid * EW + g * EC
            pltpu.sync_copy(src_h.at[pl.ds(base, EC)], sidx)
            pltpu.sync_copy(dst_h.at[pl.ds(base, EC)], didx)
            c1 = pltpu.async_copy(s_h.at[sidx], gs, sem1)
            c2 = pltpu.async_copy(d_h.at[didx], gd, sem2)
            c1.wait()
            c2.wait()

            @pl.loop(0, EC, unroll=4)
            def _j(j):
                a = gs[j] + gd[j]
                a = jnp.where(a >= 0, a, 0.2 * a)
                exb[j] = jnp.exp(a)

            pltpu.sync_copy(exb, ex_h.at[pl.ds(base, EC)])
            pltpu.sync_copy(exb, acc.at[didx], add=True)

        _flush_acc(acc, den_h)

    return k(src, dst, s_tab, d_tab)


def _sc_pass_b(src, dst, ex, htab, pslice):
    @functools.partial(
        pl.kernel,
        mesh=plsc.VectorSubcoreMesh(**_MESH),
        compiler_params=pltpu.CompilerParams(use_tc_tiling_on_sc=False),
        out_type=jax.ShapeDtypeStruct((2, NT, 16), _F32),
        scratch_types=[
            pltpu.VMEM((EC,), jnp.int32), pltpu.VMEM((EC,), jnp.int32),
            pltpu.VMEM((EC, 16), _F32), pltpu.VMEM((EC, 16), _F32),
            pltpu.VMEM((EC, 16), _F32),
            pltpu.VMEM_SHARED((NT, 16), _F32),
            pltpu.SemaphoreType.DMA, pltpu.SemaphoreType.DMA,
        ],
    )
    def k(src_h, dst_h, ex_h, h_h, agg_h,
          sidx, didx, exb, gh, msgb, acc, sem1, sem2):
        wid = _worker_id()
        _zero_acc(msgb, acc)

        it = lax.iota(jnp.int32, 16)
        if pslice is not None:
            f0 = 16 * pslice
            hb = f0 // 10
            b1 = (hb + 1) * 10 - f0
            hl = it * 0 + hb + jnp.where(it >= b1, 1, 0)
            if b1 + 10 <= 15:
                hl = hl + jnp.where(it >= b1 + 10, 1, 0)
            hl2 = hl.reshape(16, 1)
            _dn = lax.GatherDimensionNumbers(
                offset_dims=(), collapsed_slice_dims=(0,),
                start_index_map=(0,))

        @pl.loop(0, NCH)
        def _g(g):
            base = wid * EW + g * EC
            pltpu.sync_copy(src_h.at[pl.ds(base, EC)], sidx)
            pltpu.sync_copy(dst_h.at[pl.ds(base, EC)], didx)
            c1 = pltpu.async_copy(h_h.at[sidx], gh, sem1)
            c2 = pltpu.async_copy(ex_h.at[pl.ds(base, EC)], exb, sem2)
            c1.wait()
            c2.wait()

            @pl.loop(0, EC, unroll=4)
            def _j(j):
                e = exb[j]
                if pslice is not None:
                    e = lax.gather(e, hl2, _dn, slice_sizes=(1,),
                                   mode=lax.GatherScatterMode.PROMISE_IN_BOUNDS)
                msgb[j] = e * gh[j]

            pltpu.sync_copy(msgb, acc.at[didx], add=True)

        _flush_acc(acc, agg_h)

    return k(src, dst, ex, htab)


_PC = 64
_PW = 65536 // 32
_PNCH = _PW // _PC


def _sc_pairs(ei0, ei1, tid, y):
    @functools.partial(
        pl.kernel,
        mesh=plsc.VectorSubcoreMesh(**_MESH),
        compiler_params=pltpu.CompilerParams(use_tc_tiling_on_sc=False),
        out_type=jax.ShapeDtypeStruct((65536, 512), _F32),
        scratch_types=[
            pltpu.VMEM((_PC,), jnp.int32), pltpu.VMEM((_PC,), jnp.int32),
            pltpu.VMEM((_PC,), jnp.int32),
            pltpu.VMEM((_PC, 512), _F32), pltpu.VMEM((_PC, 512), _F32),
            pltpu.SemaphoreType.DMA, pltpu.SemaphoreType.DMA,
        ],
    )
    def k(e0_h, e1_h, tid_h, y_h, prod_h,
          tbuf, sidx, didx, ys, yd, sem1, sem2):
        wid = _worker_id()

        @pl.loop(0, _PNCH)
        def _g(g):
            base = wid * _PW + g * _PC
            pltpu.sync_copy(tid_h.at[pl.ds(base, _PC)], tbuf)
            c1 = pltpu.async_copy(e0_h.at[tbuf], sidx, sem1)
            c2 = pltpu.async_copy(e1_h.at[tbuf], didx, sem2)
            c1.wait()
            c2.wait()
            c3 = pltpu.async_copy(y_h.at[sidx], ys, sem1)
            c4 = pltpu.async_copy(y_h.at[didx], yd, sem2)
            c3.wait()
            c4.wait()

            @pl.loop(0, _PC)
            def _j(j):
                @pl.loop(0, 32, unroll=8)
                def _v(v):
                    sl = pl.ds(v * 16, 16)
                    ys[j, sl] = ys[j, sl] * yd[j, sl]

            pltpu.sync_copy(ys, prod_h.at[pl.ds(base, _PC)])

    return k(ei0, ei1, tid, y)


def kernel(x, edge_index, train_edge_id, W_ppc, b_ppc, W1, a1_src, a1_dst, b1,
           W2, a2_src, a2_dst, b2, lin1_W, lin1_b, lin2_W, lin2_b, fc2_W, fc2_b):
    f32 = jnp.float32
    wmat = jnp.transpose(W_ppc, (2, 1, 0)).reshape(48, 256)
    u1 = jnp.einsum('chj,hj->ch', W1.reshape(256, 8, 10), a1_src)
    v1 = jnp.einsum('chj,hj->ch', W1.reshape(256, 8, 10), a1_dst)
    wbig = jnp.concatenate([W1, u1, u1, v1, v1], axis=1)
    u2 = W2 @ a2_src[0]
    v2 = W2 @ a2_dst[0]
    wsd2 = jnp.concatenate([jnp.tile(u2[:, None], (1, 16)),
                            jnp.tile(v2[:, None], (1, 16))], axis=1)
    lane = np.arange(16)[:, None]
    feat = np.arange(80)[None, :]
    e8 = jnp.asarray((lane == feat // 10).astype(np.float32))
    fw = jnp.pad(fc2_W, ((0, 0), (0, 121)))
    fb = jnp.pad(fc2_b, (0, 121)).reshape(1, 128)

    loops = jnp.arange(N, dtype=jnp.int32)
    npad = EP - E - N
    src = jnp.concatenate([edge_index[0], loops,
                           jnp.zeros((npad,), jnp.int32)])
    dst = jnp.concatenate([edge_index[1], loops,
                           jnp.full((npad,), N, jnp.int32)])

    xpad = jnp.pad(x, ((0, NPAD - N), (0, 0), (0, 0)))

    t1 = _conv_proj1(xpad, wmat, b_ppc.reshape(1, 256), wbig)
    h1_tabs, s1t, d1t = t1[:5], t1[5], t1[6]

    ex1, den1 = _sc_pass_a(src, dst, s1t, d1t)
    agg1 = [_sc_pass_b(src, dst, ex1, h1_tabs[p], p) for p in range(5)]

    t2 = _proj2(agg1, den1, b1.reshape(1, 80), e8, wsd2)
    t2 = [jnp.pad(t, ((0, NT - N), (0, 0))) for t in t2]
    x1_tabs, s2t, d2t = t2[:5], t2[5], t2[6]

    ex2, den2 = _sc_pass_a(src, dst, s2t, d2t)
    agg2 = [_sc_pass_b(src, dst, ex2, x1_tabs[p], None) for p in range(5)]

    y = _tail(agg2, den2, W2, b2.reshape(1, 512), lin1_W,
              lin1_b.reshape(1, 512), lin2_W, lin2_b.reshape(1, 512))

    prod = _sc_pairs(edge_index[0], edge_index[1], train_edge_id, y)
    out = _fc2(prod, fw, fb)
    return out[:, :7]


# speedup vs baseline: 20.2897x; 1.0342x over previous
import functools
import jax
import jax.numpy as jnp
import numpy as np
from jax import lax
from jax.experimental import pallas as pl
from jax.experimental.pallas import tpu as pltpu
from jax.experimental.pallas import tpu_sc as plsc

N = 50000
E = 800000
L = 50
NPAD = 50048
NT = NPAD
EP = 860160
EW = EP // 32
EC = 1280
NCH = EW // EC
RSUB = NT // 16


def _lrelu(v):
    return jnp.where(v >= 0, v, 0.2 * v)


def _k1_body(xb, wmat, bppc, wbig, h0, h1o, h2, h3, h4, s1o, d1o):
    Bn = xb.shape[0]
    xp = jnp.pad(xb[...], ((0, 0), (1, 1), (0, 0)))
    xcat = jnp.concatenate(
        [xp[:, 0:L, :], xp[:, 1:L + 1, :], xp[:, 2:L + 2, :]], axis=2)
    a = xcat.reshape(Bn * L, 48)
    z = jnp.dot(a, wmat[...], preferred_element_type=jnp.float32)
    p = jnp.maximum(jnp.max(z.reshape(Bn, L, 256), axis=1) + bppc[...][0], 0.0)
    m = jnp.dot(p, wbig[...], preferred_element_type=jnp.float32)
    for i, o in enumerate((h0, h1o, h2, h3, h4)):
        o[...] = m[:, 16 * i:16 * i + 16]
    s1o[...] = m[:, 80:96]
    d1o[...] = m[:, 96:112]


def _conv_proj1(xpad, wmat, bppc, wbig):
    Bn = 128
    grid = NPAD // Bn
    outs = [jax.ShapeDtypeStruct((NPAD, 16), jnp.float32)] * 7
    ospec = pl.BlockSpec((Bn, 16), lambda i: (i, 0))
    return pl.pallas_call(
        _k1_body,
        grid=(grid,),
        in_specs=[
            pl.BlockSpec((Bn, L, 16), lambda i: (i, 0, 0)),
            pl.BlockSpec((48, 256), lambda i: (0, 0)),
            pl.BlockSpec((1, 256), lambda i: (0, 0)),
            pl.BlockSpec((256, 112), lambda i: (0, 0)),
        ],
        out_specs=[ospec] * 7,
        out_shape=outs,
    )(xpad, wmat, bppc, wbig)


def _k2_body(a0, a1, a2, a3, a4, den, b1, e8, wsd2,
             x0, x1o, x2, x3, x4, s2o, d2o):
    parts = [r[...][0] + r[...][1] for r in (a0, a1, a2, a3, a4)]
    agg = jnp.concatenate(parts, axis=1)
    den16 = den[...][0] + den[...][1]
    denexp = jnp.dot(den16, e8[...], preferred_element_type=jnp.float32)
    xv = jnp.maximum(agg / (denexp + 1e-16) + b1[...], 0.0)
    m2 = jnp.dot(xv, wsd2[...], preferred_element_type=jnp.float32)
    for i, o in enumerate((x0, x1o, x2, x3, x4)):
        o[...] = xv[:, 16 * i:16 * i + 16]
    s2o[...] = m2[:, 0:16]
    d2o[...] = m2[:, 16:32]


def _proj2(aggs, den, b1, e8, wsd2):
    Bn = 2000
    grid = N // Bn
    aspec = pl.BlockSpec((2, Bn, 16), lambda i: (0, i, 0))
    ospec = pl.BlockSpec((Bn, 16), lambda i: (i, 0))
    return pl.pallas_call(
        _k2_body,
        grid=(grid,),
        in_specs=[aspec] * 5 + [
            aspec,
            pl.BlockSpec((1, 80), lambda i: (0, 0)),
            pl.BlockSpec((16, 80), lambda i: (0, 0)),
            pl.BlockSpec((80, 32), lambda i: (0, 0)),
        ],
        out_specs=[ospec] * 7,
        out_shape=[jax.ShapeDtypeStruct((N, 16), jnp.float32)] * 7,
    )(*aggs, den, b1, e8, wsd2)


def _k3_body(a0, a1, a2, a3, a4, den, w2, b2, l1w, l1b, l2w, l2b, yo):
    parts = [r[...][0] + r[...][1] for r in (a0, a1, a2, a3, a4)]
    agg = jnp.concatenate(parts, axis=1)
    den16 = den[...][0] + den[...][1]
    dcol = den16[:, 0:1] + 1e-16
    zn = agg / dcol
    z2 = jnp.dot(zn, w2[...], preferred_element_type=jnp.float32) + b2[...]
    t = jnp.maximum(
        jnp.dot(z2, l1w[...], preferred_element_type=jnp.float32) + l1b[...], 0.0)
    yo[...] = jnp.dot(t, l2w[...], preferred_element_type=jnp.float32) + l2b[...]


def _tail(aggs, den, w2, b2, l1w, l1b, l2w, l2b):
    Bn = 1000
    grid = N // Bn
    aspec = pl.BlockSpec((2, Bn, 16), lambda i: (0, i, 0))
    return pl.pallas_call(
        _k3_body,
        grid=(grid,),
        in_specs=[aspec] * 6 + [
            pl.BlockSpec((80, 512), lambda i: (0, 0)),
            pl.BlockSpec((1, 512), lambda i: (0, 0)),
            pl.BlockSpec((512, 512), lambda i: (0, 0)),
            pl.BlockSpec((1, 512), lambda i: (0, 0)),
            pl.BlockSpec((512, 512), lambda i: (0, 0)),
            pl.BlockSpec((1, 512), lambda i: (0, 0)),
        ],
        out_specs=pl.BlockSpec((Bn, 512), lambda i: (i, 0)),
        out_shape=jax.ShapeDtypeStruct((N, 512), jnp.float32),
    )(*aggs, den, w2, b2, l1w, l1b, l2w, l2b)


def _k4_body(prod, fw, fb, oo):
    oo[...] = jnp.dot(prod[...], fw[...],
                      preferred_element_type=jnp.float32) + fb[...]


def _fc2(prod, fw, fb):
    Bp = 2048
    grid = prod.shape[0] // Bp
    return pl.pallas_call(
        _k4_body,
        grid=(grid,),
        in_specs=[
            pl.BlockSpec((Bp, 512), lambda i: (i, 0)),
            pl.BlockSpec((512, 128), lambda i: (0, 0)),
            pl.BlockSpec((1, 128), lambda i: (0, 0)),
        ],
        out_specs=pl.BlockSpec((Bp, 128), lambda i: (i, 0)),
        out_shape=jax.ShapeDtypeStruct((prod.shape[0], 128), jnp.float32),
    )(prod, fw, fb)


_MESH = dict(core_axis_name="c", subcore_axis_name="s")
_F32 = jnp.float32


def _worker_id():
    return lax.axis_index("s") * 2 + lax.axis_index("c")


def _zero_acc(zb, acc):
    sid = lax.axis_index("s")

    @pl.loop(0, EC)
    def _z(j):
        zb[j] = jnp.zeros((16,), _F32)

    r0 = sid * RSUB
    off = 0
    while off < RSUB:
        n = min(EC, RSUB - off)
        pltpu.sync_copy(zb.at[pl.ds(0, n)], acc.at[pl.ds(r0 + off, n)])
        off += n
    plsc.subcore_barrier()


def _flush_acc(acc, out_h):
    plsc.subcore_barrier()
    cid = lax.axis_index("c")
    sid = lax.axis_index("s")
    r0 = sid * RSUB
    pltpu.sync_copy(acc.at[pl.ds(r0, RSUB)], out_h.at[cid, pl.ds(r0, RSUB)])


def _sc_pass_a(src, dst, s_tab, d_tab):
    @functools.partial(
        pl.kernel,
        mesh=plsc.VectorSubcoreMesh(**_MESH),
        compiler_params=pltpu.CompilerParams(use_tc_tiling_on_sc=False),
        out_type=[jax.ShapeDtypeStruct((EP, 16), _F32),
                  jax.ShapeDtypeStruct((2, NT, 16), _F32)],
        scratch_types=[
            pltpu.VMEM((EC,), jnp.int32), pltpu.VMEM((EC,), jnp.int32),
            pltpu.VMEM((EC, 16), _F32), pltpu.VMEM((EC, 16), _F32),
            pltpu.VMEM((EC, 16), _F32),
            pltpu.VMEM_SHARED((NT, 16), _F32),
            pltpu.SemaphoreType.DMA, pltpu.SemaphoreType.DMA,
        ],
    )
    def k(src_h, dst_h, s_h, d_h, ex_h, den_h,
          sidx, didx, gs, gd, exb, acc, sem1, sem2):
        wid = _worker_id()
        _zero_acc(exb, acc)

        @pl.loop(0, NCH)
        def _g(g):
            base = wid * EW + g * EC
            pltpu.sync_copy(src_h.at[pl.ds(base, EC)], sidx)
            pltpu.sync_copy(dst_h.at[pl.ds(base, EC)], didx)
            c1 = pltpu.async_copy(s_h.at[sidx], gs, sem1)
            c2 = pltpu.async_copy(d_h.at[didx], gd, sem2)
            c1.wait()
            c2.wait()

            @pl.loop(0, EC, unroll=4)
            def _j(j):
                a = gs[j] + gd[j]
                a = jnp.where(a >= 0, a, 0.2 * a)
                exb[j] = jnp.exp(a)

            pltpu.sync_copy(exb, ex_h.at[pl.ds(base, EC)])
            pltpu.sync_copy(exb, acc.at[didx], add=True)

        _flush_acc(acc, den_h)

    return k(src, dst, s_tab, d_tab)


def _sc_pass_b(src, dst, ex, htab, pslice):
    @functools.partial(
        pl.kernel,
        mesh=plsc.VectorSubcoreMesh(**_MESH),
        compiler_params=pltpu.CompilerParams(use_tc_tiling_on_sc=False),
        out_type=jax.ShapeDtypeStruct((2, NT, 16), _F32),
        scratch_types=[
            pltpu.VMEM((EC,), jnp.int32), pltpu.VMEM((EC,), jnp.int32),
            pltpu.VMEM((EC, 16), _F32), pltpu.VMEM((EC, 16), _F32),
            pltpu.VMEM((EC, 16), _F32),
            pltpu.VMEM_SHARED((NT, 16), _F32),
            pltpu.SemaphoreType.DMA, pltpu.SemaphoreType.DMA,
        ],
    )
    def k(src_h, dst_h, ex_h, h_h, agg_h,
          sidx, didx, exb, gh, msgb, acc, sem1, sem2):
        wid = _worker_id()
        _zero_acc(msgb, acc)

        it = lax.iota(jnp.int32, 16)
        if pslice is not None:
            f0 = 16 * pslice
            hb = f0 // 10
            b1 = (hb + 1) * 10 - f0
            hl = it * 0 + hb + jnp.where(it >= b1, 1, 0)
            if b1 + 10 <= 15:
                hl = hl + jnp.where(it >= b1 + 10, 1, 0)
            hl2 = hl.reshape(16, 1)
            _dn = lax.GatherDimensionNumbers(
                offset_dims=(), collapsed_slice_dims=(0,),
                start_index_map=(0,))

        @pl.loop(0, NCH)
        def _g(g):
            base = wid * EW + g * EC
            pltpu.sync_copy(src_h.at[pl.ds(base, EC)], sidx)
            pltpu.sync_copy(dst_h.at[pl.ds(base, EC)], didx)
            c1 = pltpu.async_copy(h_h.at[sidx], gh, sem1)
            c2 = pltpu.async_copy(ex_h.at[pl.ds(base, EC)], exb, sem2)
            c1.wait()
            c2.wait()

            @pl.loop(0, EC, unroll=4)
            def _j(j):
                e = exb[j]
                if pslice is not None:
                    e = lax.gather(e, hl2, _dn, slice_sizes=(1,),
                                   mode=lax.GatherScatterMode.PROMISE_IN_BOUNDS)
                msgb[j] = e * gh[j]

            pltpu.sync_copy(msgb, acc.at[didx], add=True)

        _flush_acc(acc, agg_h)

    return k(src, dst, ex, htab)


_PC = 64
_PW = 65536 // 32
_PNCH = _PW // _PC


def _sc_pairs(ei0, ei1, tid, y):
    @functools.partial(
        pl.kernel,
        mesh=plsc.VectorSubcoreMesh(**_MESH),
        compiler_params=pltpu.CompilerParams(use_tc_tiling_on_sc=False),
        out_type=jax.ShapeDtypeStruct((65536, 512), _F32),
        scratch_types=[
            pltpu.VMEM((_PC,), jnp.int32), pltpu.VMEM((_PC,), jnp.int32),
            pltpu.VMEM((_PC,), jnp.int32),
            pltpu.VMEM((_PC, 512), _F32), pltpu.VMEM((_PC, 512), _F32),
            pltpu.SemaphoreType.DMA, pltpu.SemaphoreType.DMA,
        ],
    )
    def k(e0_h, e1_h, tid_h, y_h, prod_h,
          tbuf, sidx, didx, ys, yd, sem1, sem2):
        wid = _worker_id()

        @pl.loop(0, _PNCH)
        def _g(g):
            base = wid * _PW + g * _PC
            pltpu.sync_copy(tid_h.at[pl.ds(base, _PC)], tbuf)
            c1 = pltpu.async_copy(e0_h.at[tbuf], sidx, sem1)
            c2 = pltpu.async_copy(e1_h.at[tbuf], didx, sem2)
            c1.wait()
            c2.wait()
            c3 = pltpu.async_copy(y_h.at[sidx], ys, sem1)
            c4 = pltpu.async_copy(y_h.at[didx], yd, sem2)
            c3.wait()
            c4.wait()

            @pl.loop(0, _PC)
            def _j(j):
                @pl.loop(0, 32, unroll=8)
                def _v(v):
                    sl = pl.ds(v * 16, 16)
                    ys[j, sl] = ys[j, sl] * yd[j, sl]

            pltpu.sync_copy(ys, prod_h.at[pl.ds(base, _PC)])

    return k(ei0, ei1, tid, y)


def kernel(x, edge_index, train_edge_id, W_ppc, b_ppc, W1, a1_src, a1_dst, b1,
           W2, a2_src, a2_dst, b2, lin1_W, lin1_b, lin2_W, lin2_b, fc2_W, fc2_b):
    f32 = jnp.float32
    wmat = jnp.transpose(W_ppc, (2, 1, 0)).reshape(48, 256)
    u1 = jnp.einsum('chj,hj->ch', W1.reshape(256, 8, 10), a1_src)
    v1 = jnp.einsum('chj,hj->ch', W1.reshape(256, 8, 10), a1_dst)
    wbig = jnp.concatenate([W1, u1, u1, v1, v1], axis=1)
    u2 = W2 @ a2_src[0]
    v2 = W2 @ a2_dst[0]
    wsd2 = jnp.concatenate([jnp.tile(u2[:, None], (1, 16)),
                            jnp.tile(v2[:, None], (1, 16))], axis=1)
    lane = np.arange(16)[:, None]
    feat = np.arange(80)[None, :]
    e8 = jnp.asarray((lane == feat // 10).astype(np.float32))
    fw = jnp.pad(fc2_W, ((0, 0), (0, 121)))
    fb = jnp.pad(fc2_b, (0, 121)).reshape(1, 128)

    loops = jnp.arange(N, dtype=jnp.int32)
    npad = EP - E - N
    src = jnp.concatenate([edge_index[0], loops,
                           jnp.zeros((npad,), jnp.int32)])
    dst = jnp.concatenate([edge_index[1], loops,
                           jnp.full((npad,), N, jnp.int32)])

    xpad = jnp.pad(x, ((0, NPAD - N), (0, 0), (0, 0)))

    t1 = _conv_proj1(xpad, wmat, b_ppc.reshape(1, 256), wbig)
    h1_tabs, s1t, d1t = t1[:5], t1[5], t1[6]

    ex1, den1 = _sc_pass_a(src, dst, s1t, d1t)
    agg1 = [_sc_pass_b(src, dst, ex1, h1_tabs[p], p) for p in range(5)]

    t2 = _proj2(agg1, den1, b1.reshape(1, 80), e8, wsd2)
    t2 = [jnp.pad(t, ((0, NT - N), (0, 0))) for t in t2]
    x1_tabs, s2t, d2t = t2[:5], t2[5], t2[6]

    ex2, den2 = _sc_pass_a(src, dst, s2t, d2t)
    agg2 = [_sc_pass_b(src, dst, ex2, x1_tabs[p], None) for p in range(5)]

    y = _tail(agg2, den2, W2, b2.reshape(1, 512), lin1_W,
              lin1_b.reshape(1, 512), lin2_W, lin2_b.reshape(1, 512))

    prod = _sc_pairs(edge_index[0], edge_index[1], train_edge_id, y)
    out = _fc2(prod, fw, fb)
    return out[:, :7]


# speedup vs baseline: 21.0878x; 1.0393x over previous
import functools
import jax
import jax.numpy as jnp
import numpy as np
from jax import lax
from jax.experimental import pallas as pl
from jax.experimental.pallas import tpu as pltpu
from jax.experimental.pallas import tpu_sc as plsc

N = 50000
E = 800000
L = 50
NPAD = 50048
NT = NPAD
EP = 860160
EW = EP // 32
EC = 1280
NCH = EW // EC
RSUB = NT // 16


def _lrelu(v):
    return jnp.where(v >= 0, v, 0.2 * v)


def _k1_body(xb, wmat, bppc, wbig, h0, h1o, h2, h3, h4, s1o, d1o):
    Bn = xb.shape[0]
    xp = jnp.pad(xb[...], ((0, 0), (1, 1), (0, 0)))
    xcat = jnp.concatenate(
        [xp[:, 0:L, :], xp[:, 1:L + 1, :], xp[:, 2:L + 2, :]], axis=2)
    a = xcat.reshape(Bn * L, 48)
    z = jnp.dot(a, wmat[...], preferred_element_type=jnp.float32)
    p = jnp.maximum(jnp.max(z.reshape(Bn, L, 256), axis=1) + bppc[...][0], 0.0)
    m = jnp.dot(p, wbig[...], preferred_element_type=jnp.float32)
    for i, o in enumerate((h0, h1o, h2, h3, h4)):
        o[...] = m[:, 16 * i:16 * i + 16]
    s1o[...] = m[:, 80:96]
    d1o[...] = m[:, 96:112]


def _conv_proj1(xin, wmat, bppc, wbig):
    Bn = 128
    grid = NPAD // Bn
    outs = [jax.ShapeDtypeStruct((NPAD, 16), jnp.float32)] * 7
    ospec = pl.BlockSpec((Bn, 16), lambda i: (i, 0))
    return pl.pallas_call(
        _k1_body,
        grid=(grid,),
        in_specs=[
            pl.BlockSpec((Bn, L, 16), lambda i: (i, 0, 0)),
            pl.BlockSpec((48, 256), lambda i: (0, 0)),
            pl.BlockSpec((1, 256), lambda i: (0, 0)),
            pl.BlockSpec((256, 112), lambda i: (0, 0)),
        ],
        out_specs=[ospec] * 7,
        out_shape=outs,
    )(xin, wmat, bppc, wbig)


def _k2_body(a0, a1, a2, a3, a4, den, b1, e8, wsd2,
             x0, x1o, x2, x3, x4, s2o, d2o):
    parts = [r[...][0] + r[...][1] for r in (a0, a1, a2, a3, a4)]
    agg = jnp.concatenate(parts, axis=1)
    den16 = den[...][0] + den[...][1]
    denexp = jnp.dot(den16, e8[...], preferred_element_type=jnp.float32)
    xv = jnp.maximum(agg / (denexp + 1e-16) + b1[...], 0.0)
    m2 = jnp.dot(xv, wsd2[...], preferred_element_type=jnp.float32)
    for i, o in enumerate((x0, x1o, x2, x3, x4)):
        o[...] = xv[:, 16 * i:16 * i + 16]
    s2o[...] = m2[:, 0:16]
    d2o[...] = m2[:, 16:32]


def _proj2(aggs, den, b1, e8, wsd2):
    Bn = 2000
    grid = N // Bn
    aspec = pl.BlockSpec((2, Bn, 16), lambda i: (0, i, 0))
    ospec = pl.BlockSpec((Bn, 16), lambda i: (i, 0))
    return pl.pallas_call(
        _k2_body,
        grid=(grid,),
        in_specs=[aspec] * 5 + [
            aspec,
            pl.BlockSpec((1, 80), lambda i: (0, 0)),
            pl.BlockSpec((16, 80), lambda i: (0, 0)),
            pl.BlockSpec((80, 32), lambda i: (0, 0)),
        ],
        out_specs=[ospec] * 7,
        out_shape=[jax.ShapeDtypeStruct((N, 16), jnp.float32)] * 7,
    )(*aggs, den, b1, e8, wsd2)


def _k3_body(a0, a1, a2, a3, a4, den, w2, b2, l1w, l1b, l2w, l2b, yo):
    parts = [r[...][0] + r[...][1] for r in (a0, a1, a2, a3, a4)]
    agg = jnp.concatenate(parts, axis=1)
    den16 = den[...][0] + den[...][1]
    dcol = den16[:, 0:1] + 1e-16
    zn = agg / dcol
    z2 = jnp.dot(zn, w2[...], preferred_element_type=jnp.float32) + b2[...]
    t = jnp.maximum(
        jnp.dot(z2, l1w[...], preferred_element_type=jnp.float32) + l1b[...], 0.0)
    yo[...] = jnp.dot(t, l2w[...], preferred_element_type=jnp.float32) + l2b[...]


def _tail(aggs, den, w2, b2, l1w, l1b, l2w, l2b):
    Bn = 1000
    grid = N // Bn
    aspec = pl.BlockSpec((2, Bn, 16), lambda i: (0, i, 0))
    return pl.pallas_call(
        _k3_body,
        grid=(grid,),
        in_specs=[aspec] * 6 + [
            pl.BlockSpec((80, 512), lambda i: (0, 0)),
            pl.BlockSpec((1, 512), lambda i: (0, 0)),
            pl.BlockSpec((512, 512), lambda i: (0, 0)),
            pl.BlockSpec((1, 512), lambda i: (0, 0)),
            pl.BlockSpec((512, 512), lambda i: (0, 0)),
            pl.BlockSpec((1, 512), lambda i: (0, 0)),
        ],
        out_specs=pl.BlockSpec((Bn, 512), lambda i: (i, 0)),
        out_shape=jax.ShapeDtypeStruct((N, 512), jnp.float32),
    )(*aggs, den, w2, b2, l1w, l1b, l2w, l2b)


def _k4_body(prod, fw, fb, oo):
    oo[...] = jnp.dot(prod[...], fw[...],
                      preferred_element_type=jnp.float32) + fb[...]


def _fc2(prod, fw, fb):
    Bp = 2048
    grid = prod.shape[0] // Bp
    return pl.pallas_call(
        _k4_body,
        grid=(grid,),
        in_specs=[
            pl.BlockSpec((Bp, 512), lambda i: (i, 0)),
            pl.BlockSpec((512, 128), lambda i: (0, 0)),
            pl.BlockSpec((1, 128), lambda i: (0, 0)),
        ],
        out_specs=pl.BlockSpec((Bp, 128), lambda i: (i, 0)),
        out_shape=jax.ShapeDtypeStruct((prod.shape[0], 128), jnp.float32),
    )(prod, fw, fb)


_MESH = dict(core_axis_name="c", subcore_axis_name="s")
_F32 = jnp.float32


def _worker_id():
    return lax.axis_index("s") * 2 + lax.axis_index("c")


def _zero_acc(zb, acc):
    sid = lax.axis_index("s")

    @pl.loop(0, EC)
    def _z(j):
        zb[j] = jnp.zeros((16,), _F32)

    r0 = sid * RSUB
    off = 0
    while off < RSUB:
        n = min(EC, RSUB - off)
        pltpu.sync_copy(zb.at[pl.ds(0, n)], acc.at[pl.ds(r0 + off, n)])
        off += n
    plsc.subcore_barrier()


def _flush_acc(acc, out_h):
    plsc.subcore_barrier()
    cid = lax.axis_index("c")
    sid = lax.axis_index("s")
    r0 = sid * RSUB
    pltpu.sync_copy(acc.at[pl.ds(r0, RSUB)], out_h.at[cid, pl.ds(r0, RSUB)])


def _sc_pass_a(src, dst, s_tab, d_tab):
    @functools.partial(
        pl.kernel,
        mesh=plsc.VectorSubcoreMesh(**_MESH),
        compiler_params=pltpu.CompilerParams(use_tc_tiling_on_sc=False),
        out_type=[jax.ShapeDtypeStruct((EP, 16), _F32),
                  jax.ShapeDtypeStruct((2, NT, 16), _F32)],
        scratch_types=[
            pltpu.VMEM((EC,), jnp.int32), pltpu.VMEM((EC,), jnp.int32),
            pltpu.VMEM((EC, 16), _F32), pltpu.VMEM((EC, 16), _F32),
            pltpu.VMEM((EC, 16), _F32),
            pltpu.VMEM_SHARED((NT, 16), _F32),
            pltpu.SemaphoreType.DMA, pltpu.SemaphoreType.DMA,
        ],
    )
    def k(src_h, dst_h, s_h, d_h, ex_h, den_h,
          sidx, didx, gs, gd, exb, acc, sem1, sem2):
        wid = _worker_id()
        _zero_acc(exb, acc)

        @pl.loop(0, NCH)
        def _g(g):
            base = wid * EW + g * EC
            pltpu.sync_copy(src_h.at[pl.ds(base, EC)], sidx)
            pltpu.sync_copy(dst_h.at[pl.ds(base, EC)], didx)
            c1 = pltpu.async_copy(s_h.at[sidx], gs, sem1)
            c2 = pltpu.async_copy(d_h.at[didx], gd, sem2)
            c1.wait()
            c2.wait()

            @pl.loop(0, EC, unroll=4)
            def _j(j):
                a = gs[j] + gd[j]
                a = jnp.where(a >= 0, a, 0.2 * a)
                exb[j] = jnp.exp(a)

            pltpu.sync_copy(exb, ex_h.at[pl.ds(base, EC)])
            pltpu.sync_copy(exb, acc.at[didx], add=True)

        _flush_acc(acc, den_h)

    return k(src, dst, s_tab, d_tab)


def _sc_pass_b(src, dst, ex, htab, pslice):
    @functools.partial(
        pl.kernel,
        mesh=plsc.VectorSubcoreMesh(**_MESH),
        compiler_params=pltpu.CompilerParams(use_tc_tiling_on_sc=False),
        out_type=jax.ShapeDtypeStruct((2, NT, 16), _F32),
        scratch_types=[
            pltpu.VMEM((EC,), jnp.int32), pltpu.VMEM((EC,), jnp.int32),
            pltpu.VMEM((EC, 16), _F32), pltpu.VMEM((EC, 16), _F32),
            pltpu.VMEM((EC, 16), _F32),
            pltpu.VMEM_SHARED((NT, 16), _F32),
            pltpu.SemaphoreType.DMA, pltpu.SemaphoreType.DMA,
        ],
    )
    def k(src_h, dst_h, ex_h, h_h, agg_h,
          sidx, didx, exb, gh, msgb, acc, sem1, sem2):
        wid = _worker_id()
        _zero_acc(msgb, acc)

        it = lax.iota(jnp.int32, 16)
        if pslice is not None:
            f0 = 16 * pslice
            hb = f0 // 10
            b1 = (hb + 1) * 10 - f0
            hl = it * 0 + hb + jnp.where(it >= b1, 1, 0)
            if b1 + 10 <= 15:
                hl = hl + jnp.where(it >= b1 + 10, 1, 0)
            hl2 = hl.reshape(16, 1)
            _dn = lax.GatherDimensionNumbers(
                offset_dims=(), collapsed_slice_dims=(0,),
                start_index_map=(0,))

        @pl.loop(0, NCH)
        def _g(g):
            base = wid * EW + g * EC
            pltpu.sync_copy(src_h.at[pl.ds(base, EC)], sidx)
            pltpu.sync_copy(dst_h.at[pl.ds(base, EC)], didx)
            c1 = pltpu.async_copy(h_h.at[sidx], gh, sem1)
            c2 = pltpu.async_copy(ex_h.at[pl.ds(base, EC)], exb, sem2)
            c1.wait()
            c2.wait()

            @pl.loop(0, EC, unroll=4)
            def _j(j):
                e = exb[j]
                if pslice is not None:
                    e = lax.gather(e, hl2, _dn, slice_sizes=(1,),
                                   mode=lax.GatherScatterMode.PROMISE_IN_BOUNDS)
                msgb[j] = e * gh[j]

            pltpu.sync_copy(msgb, acc.at[didx], add=True)

        _flush_acc(acc, agg_h)

    return k(src, dst, ex, htab)


_PC = 64
_PW = 65536 // 32
_PNCH = _PW // _PC


def _sc_pairs(ei0, ei1, tid, y):
    @functools.partial(
        pl.kernel,
        mesh=plsc.VectorSubcoreMesh(**_MESH),
        compiler_params=pltpu.CompilerParams(use_tc_tiling_on_sc=False),
        out_type=jax.ShapeDtypeStruct((65536, 512), _F32),
        scratch_types=[
            pltpu.VMEM((_PC,), jnp.int32), pltpu.VMEM((_PC,), jnp.int32),
            pltpu.VMEM((_PC,), jnp.int32),
            pltpu.VMEM((_PC, 512), _F32), pltpu.VMEM((_PC, 512), _F32),
            pltpu.SemaphoreType.DMA, pltpu.SemaphoreType.DMA,
        ],
    )
    def k(e0_h, e1_h, tid_h, y_h, prod_h,
          tbuf, sidx, didx, ys, yd, sem1, sem2):
        wid = _worker_id()

        @pl.loop(0, _PNCH)
        def _g(g):
            base = wid * _PW + g * _PC
            pltpu.sync_copy(tid_h.at[pl.ds(base, _PC)], tbuf)
            c1 = pltpu.async_copy(e0_h.at[tbuf], sidx, sem1)
            c2 = pltpu.async_copy(e1_h.at[tbuf], didx, sem2)
            c1.wait()
            c2.wait()
            c3 = pltpu.async_copy(y_h.at[sidx], ys, sem1)
            c4 = pltpu.async_copy(y_h.at[didx], yd, sem2)
            c3.wait()
            c4.wait()

            @pl.loop(0, _PC)
            def _j(j):
                @pl.loop(0, 32, unroll=8)
                def _v(v):
                    sl = pl.ds(v * 16, 16)
                    ys[j, sl] = ys[j, sl] * yd[j, sl]

            pltpu.sync_copy(ys, prod_h.at[pl.ds(base, _PC)])

    return k(ei0, ei1, tid, y)


def kernel(x, edge_index, train_edge_id, W_ppc, b_ppc, W1, a1_src, a1_dst, b1,
           W2, a2_src, a2_dst, b2, lin1_W, lin1_b, lin2_W, lin2_b, fc2_W, fc2_b):
    f32 = jnp.float32
    wmat = jnp.transpose(W_ppc, (2, 1, 0)).reshape(48, 256)
    u1 = jnp.einsum('chj,hj->ch', W1.reshape(256, 8, 10), a1_src)
    v1 = jnp.einsum('chj,hj->ch', W1.reshape(256, 8, 10), a1_dst)
    wbig = jnp.concatenate([W1, u1, u1, v1, v1], axis=1)
    u2 = W2 @ a2_src[0]
    v2 = W2 @ a2_dst[0]
    wsd2 = jnp.concatenate([jnp.tile(u2[:, None], (1, 16)),
                            jnp.tile(v2[:, None], (1, 16))], axis=1)
    lane = np.arange(16)[:, None]
    feat = np.arange(80)[None, :]
    e8 = jnp.asarray((lane == feat // 10).astype(np.float32))
    fw = jnp.pad(fc2_W, ((0, 0), (0, 121)))
    fb = jnp.pad(fc2_b, (0, 121)).reshape(1, 128)

    loops = jnp.arange(N, dtype=jnp.int32)
    npad = EP - E - N
    src = jnp.concatenate([edge_index[0], loops,
                           jnp.zeros((npad,), jnp.int32)])
    dst = jnp.concatenate([edge_index[1], loops,
                           jnp.full((npad,), N, jnp.int32)])

    t1 = _conv_proj1(x, wmat, b_ppc.reshape(1, 256), wbig)
    h1_tabs, s1t, d1t = t1[:5], t1[5], t1[6]

    ex1, den1 = _sc_pass_a(src, dst, s1t, d1t)
    agg1 = [_sc_pass_b(src, dst, ex1, h1_tabs[p], p) for p in range(5)]

    t2 = _proj2(agg1, den1, b1.reshape(1, 80), e8, wsd2)
    t2 = [jnp.pad(t, ((0, NT - N), (0, 0))) for t in t2]
    x1_tabs, s2t, d2t = t2[:5], t2[5], t2[6]

    ex2, den2 = _sc_pass_a(src, dst, s2t, d2t)
    agg2 = [_sc_pass_b(src, dst, ex2, x1_tabs[p], None) for p in range(5)]

    y = _tail(agg2, den2, W2, b2.reshape(1, 512), lin1_W,
              lin1_b.reshape(1, 512), lin2_W, lin2_b.reshape(1, 512))

    prod = _sc_pairs(edge_index[0], edge_index[1], train_edge_id, y)
    out = _fc2(prod, fw, fb)
    return out[:, :7]
